# Initial kernel scaffold; baseline (speedup 1.0000x reference)
import functools

import jax
import jax.numpy as jnp
from jax import lax
from jax.experimental import pallas as pl
from jax.experimental.pallas import tpu as pltpu
from jax.experimental.pallas import tpu_sc as plsc

NUM_EMB = 1_000_000
NUM_SUB = 8
CB_SIZE = 256
SUB_DIM = 16
EMB_DIM = NUM_SUB * SUB_DIM

N_TOKENS = 4096 * 50
NC, NS = 2, 16
NW = NC * NS
CHUNK = 128
PER_W = N_TOKENS // NW
N_CHUNKS = PER_W // CHUNK
ROWS = CHUNK * NUM_SUB


def _pq_body(ids_hbm, cb_hbm, codes_hbm, out_hbm,
             ids_v, codes_v, fidx_v, out_v, cb_sh, sem):
    cid = lax.axis_index("c")
    sid = lax.axis_index("s")
    wid = sid * NC + cid

    @pl.when(sid == 0)
    def _():
        pltpu.sync_copy(cb_hbm, cb_sh)

    plsc.subcore_barrier()

    iota = lax.iota(jnp.int32, 16)
    lane_div8 = iota // 8
    lane_mod8 = iota % 8
    bias = lane_mod8 * CB_SIZE

    def chunk_body(g, carry):
        base = (wid * N_CHUNKS + g) * CHUNK
        pltpu.sync_copy(ids_hbm.at[pl.ds(base, CHUNK)], ids_v)
        pltpu.async_copy(codes_hbm.at[ids_v], codes_v, sem).wait()
        for i in range(ROWS // 16):
            row = lane_div8 + (2 * i)
            code = plsc.load_gather(codes_v, [row, lane_mod8])
            fidx_v[i // 8, pl.ds((i % 8) * 16, 16)] = code + bias
        copies = [
            pltpu.async_copy(cb_sh.at[fidx_v.at[j]],
                             out_v.at[pl.ds(j * CHUNK, CHUNK)], sem)
            for j in range(NUM_SUB)
        ]
        for c in copies:
            c.wait()
        pltpu.sync_copy(out_v, out_hbm.at[pl.ds(base * NUM_SUB, ROWS)])
        return carry

    lax.fori_loop(0, N_CHUNKS, chunk_body, 0)


@jax.jit
def _pq_lookup(ids_flat, cb_flat, codes):
    mesh = plsc.VectorSubcoreMesh(core_axis_name="c", subcore_axis_name="s")
    run = pl.kernel(
        _pq_body,
        out_type=jax.ShapeDtypeStruct((N_TOKENS * NUM_SUB, SUB_DIM),
                                      jnp.float32),
        mesh=mesh,
        scratch_types=[
            pltpu.VMEM((CHUNK,), jnp.int32),
            pltpu.VMEM((CHUNK, NUM_SUB), jnp.int32),
            pltpu.VMEM((NUM_SUB, CHUNK), jnp.int32),
            pltpu.VMEM((ROWS, SUB_DIM), jnp.float32),
            pltpu.VMEM_SHARED((NUM_SUB * CB_SIZE, SUB_DIM), jnp.float32),
            pltpu.SemaphoreType.DMA,
        ],
    )
    return run(ids_flat, cb_flat, codes)


def kernel(input_ids, codebooks, codes):
    ids_flat = input_ids.reshape(-1).astype(jnp.int32)
    cb_flat = codebooks.reshape(NUM_SUB * CB_SIZE, SUB_DIM)
    out = _pq_lookup(ids_flat, cb_flat, codes)
    return out.reshape(input_ids.shape[0], input_ids.shape[1], EMB_DIM)


# speedup vs baseline: 9.0425x; 9.0425x over previous
import functools

import jax
import jax.numpy as jnp
from jax import lax
from jax.experimental import pallas as pl
from jax.experimental.pallas import tpu as pltpu
from jax.experimental.pallas import tpu_sc as plsc

NUM_EMB = 1_000_000
NUM_SUB = 8
CB_SIZE = 256
SUB_DIM = 16
EMB_DIM = NUM_SUB * SUB_DIM

N_TOKENS = 4096 * 50
NC, NS = 2, 16
NW = NC * NS
CHUNK = 128
PER_W = N_TOKENS // NW
N_CHUNKS = PER_W // CHUNK
ROWS = CHUNK * NUM_SUB


def _pq_body(ids_hbm, cb_hbm, codes_hbm, out_hbm,
             ids_v, codes_v, fidx_v, out_v, cb_sh, sem):
    cid = lax.axis_index("c")
    sid = lax.axis_index("s")
    wid = sid * NC + cid

    @pl.when(sid == 0)
    def _():
        pltpu.sync_copy(cb_hbm, cb_sh)

    plsc.subcore_barrier()

    def chunk_body(g, carry):
        iota = lax.iota(jnp.int32, 16)
        lane_div8 = iota // 8
        lane_mod8 = iota % 8
        bias = lane_mod8 * CB_SIZE
        base = (wid * N_CHUNKS + g) * CHUNK
        pltpu.sync_copy(ids_hbm.at[pl.ds(base, CHUNK)], ids_v)
        pltpu.async_copy(codes_hbm.at[ids_v], codes_v, sem).wait()
        for i in range(ROWS // 16):
            row = lane_div8 + (2 * i)
            code = plsc.load_gather(codes_v, [row, lane_mod8])
            fidx_v[i // 8, pl.ds((i % 8) * 16, 16)] = code + bias
        copies = [
            pltpu.async_copy(cb_sh.at[fidx_v.at[j]],
                             out_v.at[pl.ds(j * CHUNK, CHUNK)], sem)
            for j in range(NUM_SUB)
        ]
        for c in copies:
            c.wait()
        pltpu.sync_copy(out_v, out_hbm.at[pl.ds(base * NUM_SUB, ROWS)])
        return carry

    lax.fori_loop(0, N_CHUNKS, chunk_body, 0)


@jax.jit
def _pq_lookup(ids_flat, cb_flat, codes):
    mesh = plsc.VectorSubcoreMesh(core_axis_name="c", subcore_axis_name="s")
    run = pl.kernel(
        _pq_body,
        out_type=jax.ShapeDtypeStruct((N_TOKENS * NUM_SUB, SUB_DIM),
                                      jnp.float32),
        mesh=mesh,
        compiler_params=pltpu.CompilerParams(use_tc_tiling_on_sc=False,
                                            needs_layout_passes=False),
        scratch_types=[
            pltpu.VMEM((CHUNK,), jnp.int32),
            pltpu.VMEM((CHUNK, NUM_SUB), jnp.int32),
            pltpu.VMEM((NUM_SUB, CHUNK), jnp.int32),
            pltpu.VMEM((ROWS, SUB_DIM), jnp.float32),
            pltpu.VMEM_SHARED((NUM_SUB * CB_SIZE, SUB_DIM), jnp.float32),
            pltpu.SemaphoreType.DMA,
        ],
    )
    return run(ids_flat, cb_flat, codes)


def kernel(input_ids, codebooks, codes):
    ids_flat = input_ids.reshape(-1).astype(jnp.int32)
    cb_flat = codebooks.reshape(NUM_SUB * CB_SIZE, SUB_DIM)
    out = _pq_lookup(ids_flat, cb_flat, codes)
    return out.reshape(input_ids.shape[0], input_ids.shape[1], EMB_DIM)


# speedup vs baseline: 11.9760x; 1.3244x over previous
import functools

import jax
import jax.numpy as jnp
from jax import lax
from jax.experimental import pallas as pl
from jax.experimental.pallas import tpu as pltpu
from jax.experimental.pallas import tpu_sc as plsc

NUM_EMB = 1_000_000
NUM_SUB = 8
CB_SIZE = 256
SUB_DIM = 16
EMB_DIM = NUM_SUB * SUB_DIM

N_TOKENS = 4096 * 50
NC, NS = 2, 16
NW = NC * NS
CHUNK = 128
PER_W = N_TOKENS // NW
N_CHUNKS = PER_W // CHUNK
ROWS = CHUNK * NUM_SUB


def _pq_body(ids_hbm, cb_hbm, codes_hbm, out_hbm,
             ids_v, codes_v, fidx_v, out_v, cb_sh, sem):
    cid = lax.axis_index("c")
    sid = lax.axis_index("s")
    wid = sid * NC + cid

    @pl.when(sid == 0)
    def _():
        pltpu.sync_copy(cb_hbm, cb_sh)

    plsc.subcore_barrier()

    def chunk_body(g, carry):
        iota = lax.iota(jnp.int32, 16)
        lane_div8 = iota // 8
        lane_mod8 = iota % 8
        bias = lane_mod8 * CB_SIZE
        base = (wid * N_CHUNKS + g) * CHUNK
        pltpu.sync_copy(ids_hbm.at[pl.ds(base, CHUNK)], ids_v)
        pltpu.async_copy(codes_hbm.at[ids_v], codes_v, sem).wait()
        for i in range(ROWS // 16):
            row = lane_div8 + (2 * i)
            code = plsc.load_gather(codes_v, [row, lane_mod8])
            fidx_v[i // 8, pl.ds((i % 8) * 16, 16)] = code + bias
        copies = [
            pltpu.async_copy(cb_sh.at[fidx_v.at[j]],
                             out_v.at[pl.ds(j * CHUNK, CHUNK)], sem)
            for j in range(NUM_SUB)
        ]
        for c in copies:
            c.wait()
        pltpu.sync_copy(out_v, out_hbm.at[pl.ds(base * NUM_SUB, ROWS)])
        return carry

    lax.fori_loop(0, N_CHUNKS, chunk_body, 0)


@jax.jit
def _pq_lookup(ids_flat, cb_flat, codes):
    mesh = plsc.VectorSubcoreMesh(core_axis_name="c", subcore_axis_name="s")
    run = pl.kernel(
        _pq_body,
        out_type=jax.ShapeDtypeStruct((N_TOKENS * NUM_SUB, SUB_DIM),
                                      jnp.float32),
        mesh=mesh,
        compiler_params=pltpu.CompilerParams(use_tc_tiling_on_sc=False,
                                            needs_layout_passes=False),
        scratch_types=[
            pltpu.VMEM((CHUNK,), jnp.int32),
            pltpu.VMEM((CHUNK, NUM_SUB), jnp.int32),
            pltpu.VMEM((NUM_SUB, CHUNK), jnp.int32),
            pltpu.VMEM((ROWS, SUB_DIM), jnp.float32),
            pltpu.VMEM_SHARED((NUM_SUB * CB_SIZE, SUB_DIM), jnp.float32),
            pltpu.SemaphoreType.DMA,
        ],
    )
    return run(ids_flat, cb_flat, codes)


def kernel(input_ids, codebooks, codes):
    B, L = input_ids.shape
    ids_t = input_ids.T.reshape(-1).astype(jnp.int32)
    cb_flat = codebooks.reshape(NUM_SUB * CB_SIZE, SUB_DIM)
    codes_lin = jax.lax.optimization_barrier(codes.reshape(-1))
    codes_2d = codes_lin.reshape(NUM_EMB, NUM_SUB)
    out = _pq_lookup(ids_t, cb_flat, codes_2d)
    return jnp.swapaxes(out.reshape(L, B, EMB_DIM), 0, 1)
